# Initial kernel scaffold; baseline (speedup 1.0000x reference)
import jax
import jax.numpy as jnp
from jax.experimental import pallas as pl


def kernel(x, c1w, c1b, c2w, c2b, f1w, f1b, f2w, f2b, f3w, f3b):
    raise NotImplementedError("write your pallas kernel here")


# speedup vs baseline: 6.8227x; 6.8227x over previous
import numpy as np
import jax
import jax.numpy as jnp
from jax.experimental import pallas as pl
from jax.experimental.pallas import tpu as pltpu

TB = 64
LANES = 128


def _band(kw_max, w_max, ow_max):
    t = np.zeros((kw_max, w_max, ow_max), np.float32)
    for kw in range(kw_max):
        for ow in range(ow_max):
            t[kw, ow + kw, ow] = 1.0
    return t

_T1 = jnp.asarray(_band(5, 32, 28))
_T2 = jnp.asarray(_band(5, 14, 10))

_IDX1 = jnp.asarray(np.arange(128) % 6, np.int32)
_MSK1 = jnp.asarray((np.arange(128) < 14 * 6).astype(np.float32))
_IDX2 = jnp.asarray(np.arange(128) % 16, np.int32)
_MSK2 = jnp.asarray((np.arange(128) < 5 * 16).astype(np.float32))


def _lenet_kernel(xt_ref, w1_ref, b1_ref, w2_ref, b2_ref,
                  f1_ref, f1b_ref, f2_ref, f2b_ref, f3_ref, f3b_ref, o_ref):
    tb = o_ref.shape[0]

    acc = None
    for kh in range(5):
        lhs = xt_ref[:, kh:kh + 28, :].reshape(tb * 28, 96)
        d = jnp.dot(lhs, w1_ref[kh], preferred_element_type=jnp.float32)
        acc = d if acc is None else acc + d
    y = acc.reshape(tb, 14, 2, 256)
    y = jnp.maximum(y[:, :, 0], y[:, :, 1])
    y = jnp.maximum(y[..., :LANES], y[..., LANES:])
    h1 = jnp.maximum(y + b1_ref[...], 0.0).astype(jnp.bfloat16)

    acc = None
    for kh in range(5):
        lhs = h1[:, kh:kh + 10, :].reshape(tb * 10, LANES)
        d = jnp.dot(lhs, w2_ref[kh], preferred_element_type=jnp.float32)
        acc = d if acc is None else acc + d
    y = acc.reshape(tb, 5, 2, 256)
    y = jnp.maximum(y[:, :, 0], y[:, :, 1])
    y = jnp.maximum(y[..., :LANES], y[..., LANES:])
    h2 = jnp.maximum(y + b2_ref[...], 0.0).astype(jnp.bfloat16)

    h = jnp.dot(h2[:, 0, :], f1_ref[0], preferred_element_type=jnp.float32)
    for j in range(1, 5):
        h = h + jnp.dot(h2[:, j, :], f1_ref[j], preferred_element_type=jnp.float32)
    h = jnp.maximum(h + f1b_ref[...], 0.0).astype(jnp.bfloat16)
    h = jnp.dot(h, f2_ref[...], preferred_element_type=jnp.float32)
    h = jnp.maximum(h + f2b_ref[...], 0.0).astype(jnp.bfloat16)
    h = jnp.dot(h, f3_ref[...], preferred_element_type=jnp.float32)
    o_ref[...] = h + f3b_ref[...]


def _pack_weights(c1w, c2w, f1w):
    r1 = c1w[:75, :6].astype(jnp.float32).reshape(5, 5, 3, 6)
    w1 = jnp.einsum('xwo,kxcn->kwcon', _T1, r1)
    w1 = w1.reshape(5, 32, 3, 14, 2, 6)
    w1 = jnp.transpose(w1, (0, 1, 2, 4, 3, 5)).reshape(5, 96, 2, 84)
    w1 = jnp.pad(w1, ((0, 0), (0, 0), (0, 0), (0, 128 - 84))).reshape(5, 96, 256)
    r2 = c2w[:, :, :6, :16].astype(jnp.float32)
    w2 = jnp.einsum('xwo,kxcn->kwcon', _T2, r2)
    w2 = w2.reshape(5, 14, 6, 5, 2, 16)
    w2 = jnp.transpose(w2, (0, 1, 2, 4, 3, 5)).reshape(5, 84, 2, 80)
    w2 = jnp.pad(w2, ((0, 0), (0, 128 - 84), (0, 0), (0, 128 - 80)))
    w2 = w2.reshape(5, 128, 256).astype(jnp.bfloat16)
    f1 = f1w.reshape(5, 5, 128, 128)[:, :, :16, :].reshape(5, 80, 128)
    f1 = jnp.pad(f1, ((0, 0), (0, 128 - 80), (0, 0)))
    return w1, w2, f1


def _pack_biases(c1b, c2b):
    b1 = (c1b[0, _IDX1] * _MSK1).reshape(1, 128)
    b2 = (c2b[0, _IDX2] * _MSK2).reshape(1, 128)
    return b1, b2


def kernel(x, c1w, c1b, c2w, c2b, f1w, f1b, f2w, f2b, f3w, f3b):
    B = x.shape[0]
    n_blocks = max(2, (B + TB - 1) // TB)
    Bp = n_blocks * TB
    xt = jnp.transpose(x, (0, 2, 3, 1)).reshape(B, 32, 96)
    if Bp != B:
        xt = jnp.pad(xt, ((0, Bp - B), (0, 0), (0, 0)))
    w1, w2, f1 = _pack_weights(c1w, c2w, f1w)
    b1v, b2v = _pack_biases(c1b, c2b)

    const2 = lambda i: (0, 0)
    const3 = lambda i: (0, 0, 0)
    logits = pl.pallas_call(
        _lenet_kernel,
        out_shape=jax.ShapeDtypeStruct((Bp, 128), jnp.float32),
        grid=(n_blocks,),
        in_specs=[
            pl.BlockSpec((TB, 32, 96), lambda i: (i, 0, 0)),
            pl.BlockSpec((5, 96, 256), const3),
            pl.BlockSpec((1, 128), const2),
            pl.BlockSpec((5, 128, 256), const3),
            pl.BlockSpec((1, 128), const2),
            pl.BlockSpec((5, 128, 128), const3),
            pl.BlockSpec((1, 128), const2),
            pl.BlockSpec((128, 128), const2),
            pl.BlockSpec((1, 128), const2),
            pl.BlockSpec((128, 128), const2),
            pl.BlockSpec((1, 128), const2),
        ],
        out_specs=pl.BlockSpec((TB, 128), lambda i: (i, 0)),
        compiler_params=pltpu.CompilerParams(
            dimension_semantics=("parallel",),
            vmem_limit_bytes=48 * 1024 * 1024),
    )(xt, w1, b1v, w2, b2v, f1, f1b, f2w, f2b, f3w, f3b)
    return logits[:B, :10]


# speedup vs baseline: 8.9827x; 1.3166x over previous
import numpy as np
import jax
import jax.numpy as jnp
from jax.experimental import pallas as pl
from jax.experimental.pallas import tpu as pltpu

TB = 64
LANES = 128


def _band(kw_max, w_max, ow_max):
    t = np.zeros((kw_max, w_max, ow_max), np.float32)
    for kw in range(kw_max):
        for ow in range(ow_max):
            t[kw, ow + kw, ow] = 1.0
    return t

_T1 = _band(5, 32, 28)
_T2 = _band(5, 14, 10)

_IDX1 = (np.arange(128) % 6).astype(np.int32)
_MSK1 = (np.arange(128) < 14 * 6).astype(np.float32)
_IDX2 = (np.arange(128) % 16).astype(np.int32)
_MSK2 = (np.arange(128) < 5 * 16).astype(np.float32)


def _lenet_kernel(xt_ref, w1_ref, b1_ref, w2_ref, b2_ref,
                  f1_ref, f1b_ref, f2_ref, f2b_ref, f3_ref, f3b_ref, o_ref):
    tb = o_ref.shape[0]

    acc = None
    for kh in range(5):
        lhs = xt_ref[kh:kh + 28].reshape(tb * 28, 96)
        d = jnp.dot(lhs, w1_ref[kh], preferred_element_type=jnp.float32)
        acc = d if acc is None else acc + d
    y = acc.reshape(14, 2, tb, 256)
    y = jnp.maximum(y[:, 0], y[:, 1])
    y = jnp.maximum(y[..., :LANES], y[..., LANES:])
    h1 = jnp.maximum(y + b1_ref[...], 0.0).astype(jnp.bfloat16)

    acc = None
    for kh in range(5):
        lhs = h1[kh:kh + 10].reshape(tb * 10, LANES)
        d = jnp.dot(lhs, w2_ref[kh], preferred_element_type=jnp.float32)
        acc = d if acc is None else acc + d
    y = acc.reshape(5, 2, tb, 256)
    y = jnp.maximum(y[:, 0], y[:, 1])
    y = jnp.maximum(y[..., :LANES], y[..., LANES:])
    h2 = jnp.maximum(y + b2_ref[...], 0.0).astype(jnp.bfloat16)

    h = jnp.dot(h2[0], f1_ref[0], preferred_element_type=jnp.float32)
    for j in range(1, 5):
        h = h + jnp.dot(h2[j], f1_ref[j], preferred_element_type=jnp.float32)
    h = jnp.maximum(h + f1b_ref[...], 0.0).astype(jnp.bfloat16)
    h = jnp.dot(h, f2_ref[...], preferred_element_type=jnp.float32)
    h = jnp.maximum(h + f2b_ref[...], 0.0).astype(jnp.bfloat16)
    h = jnp.dot(h, f3_ref[...], preferred_element_type=jnp.float32)
    o_ref[...] = h + f3b_ref[...]


def _pack_weights(c1w, c2w, f1w):
    r1 = c1w[:75, :6].astype(jnp.float32).reshape(5, 5, 3, 6)
    w1 = jnp.einsum('xwo,kxcn->kwcon', _T1, r1)
    w1 = w1.reshape(5, 32, 3, 14, 2, 6)
    w1 = jnp.transpose(w1, (0, 1, 2, 4, 3, 5)).reshape(5, 96, 2, 84)
    w1 = jnp.pad(w1, ((0, 0), (0, 0), (0, 0), (0, 128 - 84))).reshape(5, 96, 256)
    r2 = c2w[:, :, :6, :16].astype(jnp.float32)
    w2 = jnp.einsum('xwo,kxcn->kwcon', _T2, r2)
    w2 = w2.reshape(5, 14, 6, 5, 2, 16)
    w2 = jnp.transpose(w2, (0, 1, 2, 4, 3, 5)).reshape(5, 84, 2, 80)
    w2 = jnp.pad(w2, ((0, 0), (0, 128 - 84), (0, 0), (0, 128 - 80)))
    w2 = w2.reshape(5, 128, 256).astype(jnp.bfloat16)
    f1 = f1w.reshape(5, 5, 128, 128)[:, :, :16, :].reshape(5, 80, 128)
    f1 = jnp.pad(f1, ((0, 0), (0, 128 - 80), (0, 0)))
    return w1, w2, f1


def _pack_biases(c1b, c2b):
    b1 = (c1b[0, _IDX1] * _MSK1).reshape(1, 128)
    b2 = (c2b[0, _IDX2] * _MSK2).reshape(1, 128)
    return b1, b2


def kernel(x, c1w, c1b, c2w, c2b, f1w, f1b, f2w, f2b, f3w, f3b):
    B = x.shape[0]
    n_blocks = max(2, (B + TB - 1) // TB)
    Bp = n_blocks * TB
    xt = jnp.transpose(x, (2, 0, 3, 1)).reshape(32, B, 96)
    if Bp != B:
        xt = jnp.pad(xt, ((0, 0), (0, Bp - B), (0, 0)))
    w1, w2, f1 = _pack_weights(c1w, c2w, f1w)
    b1v, b2v = _pack_biases(c1b, c2b)

    const2 = lambda i: (0, 0)
    const3 = lambda i: (0, 0, 0)
    logits = pl.pallas_call(
        _lenet_kernel,
        out_shape=jax.ShapeDtypeStruct((Bp, 128), jnp.float32),
        grid=(n_blocks,),
        in_specs=[
            pl.BlockSpec((32, TB, 96), lambda i: (0, i, 0)),
            pl.BlockSpec((5, 96, 256), const3),
            pl.BlockSpec((1, 128), const2),
            pl.BlockSpec((5, 128, 256), const3),
            pl.BlockSpec((1, 128), const2),
            pl.BlockSpec((5, 128, 128), const3),
            pl.BlockSpec((1, 128), const2),
            pl.BlockSpec((128, 128), const2),
            pl.BlockSpec((1, 128), const2),
            pl.BlockSpec((128, 128), const2),
            pl.BlockSpec((1, 128), const2),
        ],
        out_specs=pl.BlockSpec((TB, 128), lambda i: (i, 0)),
        compiler_params=pltpu.CompilerParams(
            dimension_semantics=("parallel",),
            vmem_limit_bytes=48 * 1024 * 1024),
    )(xt, w1, b1v, w2, b2v, f1, f1b, f2w, f2b, f3w, f3b)
    return logits[:B, :10]


# speedup vs baseline: 10.6920x; 1.1903x over previous
import numpy as np
import jax
import jax.numpy as jnp
from jax.experimental import pallas as pl
from jax.experimental.pallas import tpu as pltpu

TB = 128
LANES = 128


def _band(kw_max, w_max, ow_max):
    t = np.zeros((kw_max, w_max, ow_max), np.float32)
    for kw in range(kw_max):
        for ow in range(ow_max):
            t[kw, ow + kw, ow] = 1.0
    return t

_T1 = _band(5, 32, 28)
_T2 = _band(5, 14, 10)

_IDX1 = (np.arange(128) % 6).astype(np.int32)
_MSK1 = (np.arange(128) < 14 * 6).astype(np.float32)
_IDX2 = (np.arange(128) % 16).astype(np.int32)
_MSK2 = (np.arange(128) < 5 * 16).astype(np.float32)


def _lenet_kernel(xt_ref, w1_ref, b1_ref, w2_ref, b2_ref,
                  f1_ref, f1b_ref, f2_ref, f2b_ref, f3_ref, f3b_ref, o_ref):
    tb = o_ref.shape[0]

    acc = None
    for kh in range(5):
        lhs = xt_ref[kh:kh + 28].reshape(tb * 28, 96)
        d = jnp.dot(lhs, w1_ref[kh], preferred_element_type=jnp.float32)
        acc = d if acc is None else acc + d
    y = acc.reshape(14, 2, tb, 256)
    y = jnp.maximum(y[:, 0], y[:, 1])
    y = jnp.maximum(y[..., :LANES], y[..., LANES:])
    h1 = jnp.maximum(y + b1_ref[...], 0.0).astype(jnp.bfloat16)

    acc = None
    for kh in range(5):
        lhs = h1[kh:kh + 10].reshape(tb * 10, LANES)
        d = jnp.dot(lhs, w2_ref[kh], preferred_element_type=jnp.float32)
        acc = d if acc is None else acc + d
    y = acc.reshape(5, 2, tb, 256)
    y = jnp.maximum(y[:, 0], y[:, 1])
    y = jnp.maximum(y[..., :LANES], y[..., LANES:])
    h2 = jnp.maximum(y + b2_ref[...], 0.0).astype(jnp.bfloat16)

    h = jnp.dot(h2[0], f1_ref[0], preferred_element_type=jnp.float32)
    for j in range(1, 5):
        h = h + jnp.dot(h2[j], f1_ref[j], preferred_element_type=jnp.float32)
    h = jnp.maximum(h + f1b_ref[...], 0.0).astype(jnp.bfloat16)
    h = jnp.dot(h, f2_ref[...], preferred_element_type=jnp.float32)
    h = jnp.maximum(h + f2b_ref[...], 0.0).astype(jnp.bfloat16)
    h = jnp.dot(h, f3_ref[...], preferred_element_type=jnp.float32)
    o_ref[...] = h + f3b_ref[...]


def _pack_weights(c1w, c2w, f1w):
    r1 = c1w[:75, :6].astype(jnp.float32).reshape(5, 5, 3, 6)
    w1 = jnp.einsum('xwo,kxcn->kwcon', _T1, r1)
    w1 = w1.reshape(5, 32, 3, 14, 2, 6)
    w1 = jnp.transpose(w1, (0, 1, 2, 4, 3, 5)).reshape(5, 96, 2, 84)
    w1 = jnp.pad(w1, ((0, 0), (0, 0), (0, 0), (0, 128 - 84)))
    w1 = w1.reshape(5, 96, 256).astype(jnp.bfloat16)
    r2 = c2w[:, :, :6, :16].astype(jnp.float32)
    w2 = jnp.einsum('xwo,kxcn->kwcon', _T2, r2)
    w2 = w2.reshape(5, 14, 6, 5, 2, 16)
    w2 = jnp.transpose(w2, (0, 1, 2, 4, 3, 5)).reshape(5, 84, 2, 80)
    w2 = jnp.pad(w2, ((0, 0), (0, 128 - 84), (0, 0), (0, 128 - 80)))
    w2 = w2.reshape(5, 128, 256).astype(jnp.bfloat16)
    f1 = f1w.reshape(5, 5, 128, 128)[:, :, :16, :].reshape(5, 80, 128)
    f1 = jnp.pad(f1, ((0, 0), (0, 128 - 80), (0, 0)))
    return w1, w2, f1


def _pack_biases(c1b, c2b):
    b1 = (c1b[0, _IDX1] * _MSK1).reshape(1, 128)
    b2 = (c2b[0, _IDX2] * _MSK2).reshape(1, 128)
    return b1, b2


def kernel(x, c1w, c1b, c2w, c2b, f1w, f1b, f2w, f2b, f3w, f3b):
    B = x.shape[0]
    n_blocks = max(2, (B + TB - 1) // TB)
    Bp = n_blocks * TB
    xt = jnp.transpose(x, (2, 0, 3, 1)).reshape(32, B, 96).astype(jnp.bfloat16)
    if Bp != B:
        xt = jnp.pad(xt, ((0, 0), (0, Bp - B), (0, 0)))
    w1, w2, f1 = _pack_weights(c1w, c2w, f1w)
    b1v, b2v = _pack_biases(c1b, c2b)

    const2 = lambda i: (0, 0)
    const3 = lambda i: (0, 0, 0)
    logits = pl.pallas_call(
        _lenet_kernel,
        out_shape=jax.ShapeDtypeStruct((Bp, 128), jnp.float32),
        grid=(n_blocks,),
        in_specs=[
            pl.BlockSpec((32, TB, 96), lambda i: (0, i, 0)),
            pl.BlockSpec((5, 96, 256), const3),
            pl.BlockSpec((1, 128), const2),
            pl.BlockSpec((5, 128, 256), const3),
            pl.BlockSpec((1, 128), const2),
            pl.BlockSpec((5, 128, 128), const3),
            pl.BlockSpec((1, 128), const2),
            pl.BlockSpec((128, 128), const2),
            pl.BlockSpec((1, 128), const2),
            pl.BlockSpec((128, 128), const2),
            pl.BlockSpec((1, 128), const2),
        ],
        out_specs=pl.BlockSpec((TB, 128), lambda i: (i, 0)),
        compiler_params=pltpu.CompilerParams(
            dimension_semantics=("parallel",),
            vmem_limit_bytes=48 * 1024 * 1024),
    )(xt, w1, b1v, w2, b2v, f1, f1b, f2w, f2b, f3w, f3b)
    return logits[:B, :10]


# speedup vs baseline: 13.6632x; 1.2779x over previous
import numpy as np
import jax
import jax.numpy as jnp
from jax.experimental import pallas as pl
from jax.experimental.pallas import tpu as pltpu

TB = 128
LANES = 128


def _band(kw_max, w_max, ow_max):
    t = np.zeros((kw_max, w_max, ow_max), np.float32)
    for kw in range(kw_max):
        for ow in range(ow_max):
            t[kw, ow + kw, ow] = 1.0
    return t

_T1 = _band(5, 32, 28)
_T2 = _band(5, 14, 10)

_IDX1 = (np.arange(128) % 6).astype(np.int32)
_MSK1 = (np.arange(128) < 14 * 6).astype(np.float32)
_IDX2 = (np.arange(128) % 16).astype(np.int32)
_MSK2 = (np.arange(128) < 5 * 16).astype(np.float32)


def _lenet_kernel(xt_ref, w1_ref, b1_ref, w2_ref, b2_ref,
                  f1_ref, f1b_ref, f2_ref, f2b_ref, f3_ref, f3b_ref, o_ref):
    tb = o_ref.shape[0]

    lhs = jnp.concatenate(
        [xt_ref[kh:kh + 28].reshape(tb * 28, LANES) for kh in range(5)], axis=-1)
    m1 = tb * 14
    d0 = jnp.dot(lhs[:m1], w1_ref[...], preferred_element_type=jnp.float32)
    d1 = jnp.dot(lhs[m1:], w1_ref[...], preferred_element_type=jnp.float32)
    y = jnp.concatenate([d0, d1], axis=0).reshape(14, 2, tb, 256)
    y = jnp.maximum(y[:, 0], y[:, 1])
    y = jnp.maximum(y[..., :LANES], y[..., LANES:])
    h1 = jnp.maximum(y + b1_ref[...], 0.0).astype(jnp.bfloat16)

    lhs = jnp.concatenate(
        [h1[kh:kh + 10].reshape(tb * 10, LANES) for kh in range(5)], axis=-1)
    m2 = tb * 5
    d0 = jnp.dot(lhs[:m2], w2_ref[...], preferred_element_type=jnp.float32)
    d1 = jnp.dot(lhs[m2:], w2_ref[...], preferred_element_type=jnp.float32)
    y = jnp.concatenate([d0, d1], axis=0).reshape(5, 2, tb, 256)
    y = jnp.maximum(y[:, 0], y[:, 1])
    y = jnp.maximum(y[..., :LANES], y[..., LANES:])
    h2 = jnp.maximum(y + b2_ref[...], 0.0).astype(jnp.bfloat16)

    lhs = jnp.concatenate([h2[j] for j in range(5)], axis=-1)
    h = jnp.dot(lhs, f1_ref[...], preferred_element_type=jnp.float32)
    h = jnp.maximum(h + f1b_ref[...], 0.0).astype(jnp.bfloat16)
    h = jnp.dot(h, f2_ref[...], preferred_element_type=jnp.float32)
    h = jnp.maximum(h + f2b_ref[...], 0.0).astype(jnp.bfloat16)
    h = jnp.dot(h, f3_ref[...], preferred_element_type=jnp.float32)
    o_ref[...] = h + f3b_ref[...]


def _pack_weights(c1w, c2w, f1w):
    r1 = c1w[:75, :6].astype(jnp.float32).reshape(5, 5, 3, 6)
    w1 = jnp.einsum('xwo,kxcn->kwcon', _T1, r1)
    w1 = w1.reshape(5, 32, 3, 14, 2, 6)
    w1 = jnp.transpose(w1, (0, 1, 2, 4, 3, 5)).reshape(5, 96, 2, 84)
    w1 = jnp.pad(w1, ((0, 0), (0, 0), (0, 0), (0, 128 - 84)))
    w1 = w1.reshape(5, 96, 256)
    w1 = jnp.pad(w1, ((0, 0), (0, 32), (0, 0)))
    w1 = w1.reshape(640, 256).astype(jnp.bfloat16)
    r2 = c2w[:, :, :6, :16].astype(jnp.float32)
    w2 = jnp.einsum('xwo,kxcn->kwcon', _T2, r2)
    w2 = w2.reshape(5, 14, 6, 5, 2, 16)
    w2 = jnp.transpose(w2, (0, 1, 2, 4, 3, 5)).reshape(5, 84, 2, 80)
    w2 = jnp.pad(w2, ((0, 0), (0, 128 - 84), (0, 0), (0, 128 - 80)))
    w2 = w2.reshape(640, 256).astype(jnp.bfloat16)
    f1 = f1w.reshape(5, 5, 128, 128)[:, :, :16, :].reshape(5, 80, 128)
    f1 = jnp.pad(f1, ((0, 0), (0, 128 - 80), (0, 0))).reshape(640, 128)
    return w1, w2, f1


def _pack_biases(c1b, c2b):
    b1 = (c1b[0, _IDX1] * _MSK1).reshape(1, 128)
    b2 = (c2b[0, _IDX2] * _MSK2).reshape(1, 128)
    return b1, b2


def kernel(x, c1w, c1b, c2w, c2b, f1w, f1b, f2w, f2b, f3w, f3b):
    B = x.shape[0]
    n_blocks = max(2, (B + TB - 1) // TB)
    Bp = n_blocks * TB
    xt = jnp.transpose(x, (2, 0, 3, 1)).reshape(32, B, 96).astype(jnp.bfloat16)
    xt = jnp.pad(xt, ((0, 0), (0, Bp - B), (0, 32)))
    w1, w2, f1 = _pack_weights(c1w, c2w, f1w)
    b1v, b2v = _pack_biases(c1b, c2b)

    const2 = lambda i: (0, 0)
    const3 = lambda i: (0, 0, 0)
    logits = pl.pallas_call(
        _lenet_kernel,
        out_shape=jax.ShapeDtypeStruct((Bp, 128), jnp.float32),
        grid=(n_blocks,),
        in_specs=[
            pl.BlockSpec((32, TB, 128), lambda i: (0, i, 0)),
            pl.BlockSpec((640, 256), const2),
            pl.BlockSpec((1, 128), const2),
            pl.BlockSpec((640, 256), const2),
            pl.BlockSpec((1, 128), const2),
            pl.BlockSpec((640, 128), const2),
            pl.BlockSpec((1, 128), const2),
            pl.BlockSpec((128, 128), const2),
            pl.BlockSpec((1, 128), const2),
            pl.BlockSpec((128, 128), const2),
            pl.BlockSpec((1, 128), const2),
        ],
        out_specs=pl.BlockSpec((TB, 128), lambda i: (i, 0)),
        compiler_params=pltpu.CompilerParams(
            dimension_semantics=("parallel",),
            vmem_limit_bytes=48 * 1024 * 1024),
    )(xt, w1, b1v, w2, b2v, f1, f1b, f2w, f2b, f3w, f3b)
    return logits[:B, :10]


# speedup vs baseline: 15.6564x; 1.1459x over previous
import numpy as np
import jax
import jax.numpy as jnp
from jax.experimental import pallas as pl
from jax.experimental.pallas import tpu as pltpu

TB = 128
LANES = 128


def _band(kw_max, w_max, ow_max):
    t = np.zeros((kw_max, w_max, ow_max), np.float32)
    for kw in range(kw_max):
        for ow in range(ow_max):
            t[kw, ow + kw, ow] = 1.0
    return t

_T1 = _band(5, 32, 28)
_T2 = _band(5, 14, 10)

_IDX1 = (np.arange(128) % 6).astype(np.int32)
_MSK1 = (np.arange(128) < 14 * 6).astype(np.float32)
_IDX2 = (np.arange(128) % 16).astype(np.int32)
_MSK2 = (np.arange(128) < 5 * 16).astype(np.float32)


def _lenet_kernel(xt_ref, w1_ref, b1_ref, w2_ref, b2_ref,
                  f1_ref, f1b_ref, f2_ref, f2b_ref, f3_ref, f3b_ref, o_ref):
    tb = o_ref.shape[0]

    lhs = jnp.concatenate(
        [xt_ref[kh:kh + 28].reshape(tb * 28, LANES) for kh in range(5)], axis=-1)
    m1 = tb * 14
    d0 = jnp.dot(lhs[:m1], w1_ref[...], preferred_element_type=jnp.float32)
    d1 = jnp.dot(lhs[m1:], w1_ref[...], preferred_element_type=jnp.float32)
    y = jnp.concatenate([d0, d1], axis=0).reshape(14, 2, tb, 256)
    y = jnp.maximum(y[:, 0], y[:, 1])
    y = jnp.maximum(y[..., :LANES], y[..., LANES:])
    h1 = jnp.maximum(y + b1_ref[...], 0.0).astype(jnp.bfloat16)

    lhs = jnp.concatenate(
        [h1[kh:kh + 10].reshape(tb * 10, LANES) for kh in range(5)], axis=-1)
    m2 = tb * 5
    d0 = jnp.dot(lhs[:m2], w2_ref[...], preferred_element_type=jnp.float32)
    d1 = jnp.dot(lhs[m2:], w2_ref[...], preferred_element_type=jnp.float32)
    y = jnp.concatenate([d0, d1], axis=0).reshape(5, 2, tb, 256)
    y = jnp.maximum(y[:, 0], y[:, 1])
    y = jnp.maximum(y[..., :LANES], y[..., LANES:])
    h2 = jnp.maximum(y + b2_ref[...], 0.0).astype(jnp.bfloat16)

    lhs = jnp.concatenate([h2[j] for j in range(5)], axis=-1)
    h = jnp.dot(lhs, f1_ref[...], preferred_element_type=jnp.float32)
    h = jnp.maximum(h + f1b_ref[...], 0.0).astype(jnp.bfloat16)
    h = jnp.dot(h, f2_ref[...], preferred_element_type=jnp.float32)
    h = jnp.maximum(h + f2b_ref[...], 0.0).astype(jnp.bfloat16)
    h = jnp.dot(h, f3_ref[...], preferred_element_type=jnp.float32)
    o_ref[...] = h + f3b_ref[...]


def _pack_weights(c1w, c2w, f1w):
    r1 = c1w[:75, :6].astype(jnp.float32).reshape(5, 5, 3, 6)
    w1 = jnp.einsum('xwo,kxcn->kcwon', _T1, r1)
    w1 = w1.reshape(5, 3, 32, 14, 2, 6)
    w1 = jnp.transpose(w1, (0, 1, 2, 4, 3, 5)).reshape(5, 96, 2, 84)
    w1 = jnp.pad(w1, ((0, 0), (0, 0), (0, 0), (0, 128 - 84)))
    w1 = w1.reshape(5, 96, 256)
    w1 = jnp.pad(w1, ((0, 0), (0, 32), (0, 0)))
    w1 = w1.reshape(640, 256).astype(jnp.bfloat16)
    r2 = c2w[:, :, :6, :16].astype(jnp.float32)
    w2 = jnp.einsum('xwo,kxcn->kwcon', _T2, r2)
    w2 = w2.reshape(5, 14, 6, 5, 2, 16)
    w2 = jnp.transpose(w2, (0, 1, 2, 4, 3, 5)).reshape(5, 84, 2, 80)
    w2 = jnp.pad(w2, ((0, 0), (0, 128 - 84), (0, 0), (0, 128 - 80)))
    w2 = w2.reshape(640, 256).astype(jnp.bfloat16)
    f1 = f1w.reshape(5, 5, 128, 128)[:, :, :16, :].reshape(5, 80, 128)
    f1 = jnp.pad(f1, ((0, 0), (0, 128 - 80), (0, 0))).reshape(640, 128)
    return w1, w2, f1


def _pack_biases(c1b, c2b):
    b1 = (c1b[0, _IDX1] * _MSK1).reshape(1, 128)
    b2 = (c2b[0, _IDX2] * _MSK2).reshape(1, 128)
    return b1, b2


def kernel(x, c1w, c1b, c2w, c2b, f1w, f1b, f2w, f2b, f3w, f3b):
    B = x.shape[0]
    n_blocks = max(2, (B + TB - 1) // TB)
    Bp = n_blocks * TB
    xt = jnp.transpose(x, (2, 0, 1, 3)).reshape(32, B, 96).astype(jnp.bfloat16)
    xt = jnp.pad(xt, ((0, 0), (0, Bp - B), (0, 32)))
    w1, w2, f1 = _pack_weights(c1w, c2w, f1w)
    b1v, b2v = _pack_biases(c1b, c2b)

    const2 = lambda i: (0, 0)
    const3 = lambda i: (0, 0, 0)
    logits = pl.pallas_call(
        _lenet_kernel,
        out_shape=jax.ShapeDtypeStruct((Bp, 128), jnp.float32),
        grid=(n_blocks,),
        in_specs=[
            pl.BlockSpec((32, TB, 128), lambda i: (0, i, 0)),
            pl.BlockSpec((640, 256), const2),
            pl.BlockSpec((1, 128), const2),
            pl.BlockSpec((640, 256), const2),
            pl.BlockSpec((1, 128), const2),
            pl.BlockSpec((640, 128), const2),
            pl.BlockSpec((1, 128), const2),
            pl.BlockSpec((128, 128), const2),
            pl.BlockSpec((1, 128), const2),
            pl.BlockSpec((128, 128), const2),
            pl.BlockSpec((1, 128), const2),
        ],
        out_specs=pl.BlockSpec((TB, 128), lambda i: (i, 0)),
        compiler_params=pltpu.CompilerParams(
            dimension_semantics=("parallel",),
            vmem_limit_bytes=48 * 1024 * 1024),
    )(xt, w1, b1v, w2, b2v, f1, f1b, f2w, f2b, f3w, f3b)
    return logits[:B, :10]


# speedup vs baseline: 16.4861x; 1.0530x over previous
import numpy as np
import jax
import jax.numpy as jnp
from jax.experimental import pallas as pl
from jax.experimental.pallas import tpu as pltpu

TB = 256
LANES = 128


def _band(kw_max, w_max, ow_max):
    t = np.zeros((kw_max, w_max, ow_max), np.float32)
    for kw in range(kw_max):
        for ow in range(ow_max):
            t[kw, ow + kw, ow] = 1.0
    return t

_T1 = _band(5, 32, 28)
_T2 = _band(5, 14, 10)

_IDX1 = (np.arange(128) % 6).astype(np.int32)
_MSK1 = (np.arange(128) < 14 * 6).astype(np.float32)
_IDX2 = (np.arange(128) % 16).astype(np.int32)
_MSK2 = (np.arange(128) < 5 * 16).astype(np.float32)


def _lenet_kernel(xt_ref, w1_ref, b1_ref, w2_ref, b2_ref,
                  f1_ref, f1b_ref, f2_ref, f2b_ref, f3_ref, f3b_ref, o_ref):
    tb = o_ref.shape[0]

    lhs = jnp.concatenate(
        [xt_ref[kh:kh + 28].reshape(tb * 28, LANES) for kh in range(5)], axis=-1)
    m1 = tb * 14
    d0 = jnp.dot(lhs[:m1], w1_ref[...], preferred_element_type=jnp.float32)
    d1 = jnp.dot(lhs[m1:], w1_ref[...], preferred_element_type=jnp.float32)
    y = jnp.concatenate([d0, d1], axis=0).reshape(14, 2, tb, 256)
    y = jnp.maximum(y[:, 0], y[:, 1])
    y = jnp.maximum(y[..., :LANES], y[..., LANES:])
    h1 = jnp.maximum(y + b1_ref[...], 0.0).astype(jnp.bfloat16)

    lhs = jnp.concatenate(
        [h1[kh:kh + 10].reshape(tb * 10, LANES) for kh in range(5)], axis=-1)
    m2 = tb * 5
    d0 = jnp.dot(lhs[:m2], w2_ref[...], preferred_element_type=jnp.float32)
    d1 = jnp.dot(lhs[m2:], w2_ref[...], preferred_element_type=jnp.float32)
    y = jnp.concatenate([d0, d1], axis=0).reshape(5, 2, tb, 256)
    y = jnp.maximum(y[:, 0], y[:, 1])
    y = jnp.maximum(y[..., :LANES], y[..., LANES:])
    h2 = jnp.maximum(y + b2_ref[...], 0.0).astype(jnp.bfloat16)

    lhs = jnp.concatenate([h2[j] for j in range(5)], axis=-1)
    h = jnp.dot(lhs, f1_ref[...], preferred_element_type=jnp.float32)
    h = jnp.maximum(h + f1b_ref[...], 0.0).astype(jnp.bfloat16)
    h = jnp.dot(h, f2_ref[...], preferred_element_type=jnp.float32)
    h = jnp.maximum(h + f2b_ref[...], 0.0).astype(jnp.bfloat16)
    h = jnp.dot(h, f3_ref[...], preferred_element_type=jnp.float32)
    o_ref[...] = h + f3b_ref[...]


def _pack_weights(c1w, c2w, f1w):
    r1 = c1w[:75, :6].astype(jnp.float32).reshape(5, 5, 3, 6)
    w1 = jnp.einsum('xwo,kxcn->kcwon', _T1, r1)
    w1 = w1.reshape(5, 3, 32, 14, 2, 6)
    w1 = jnp.transpose(w1, (0, 1, 2, 4, 3, 5)).reshape(5, 96, 2, 84)
    w1 = jnp.pad(w1, ((0, 0), (0, 0), (0, 0), (0, 128 - 84)))
    w1 = w1.reshape(5, 96, 256)
    w1 = jnp.pad(w1, ((0, 0), (0, 32), (0, 0)))
    w1 = w1.reshape(640, 256).astype(jnp.bfloat16)
    r2 = c2w[:, :, :6, :16].astype(jnp.float32)
    w2 = jnp.einsum('xwo,kxcn->kwcon', _T2, r2)
    w2 = w2.reshape(5, 14, 6, 5, 2, 16)
    w2 = jnp.transpose(w2, (0, 1, 2, 4, 3, 5)).reshape(5, 84, 2, 80)
    w2 = jnp.pad(w2, ((0, 0), (0, 128 - 84), (0, 0), (0, 128 - 80)))
    w2 = w2.reshape(640, 256).astype(jnp.bfloat16)
    f1 = f1w.reshape(5, 5, 128, 128)[:, :, :16, :].reshape(5, 80, 128)
    f1 = jnp.pad(f1, ((0, 0), (0, 128 - 80), (0, 0))).reshape(640, 128)
    return w1, w2, f1


def _pack_biases(c1b, c2b):
    b1 = (c1b[0, _IDX1] * _MSK1).reshape(1, 128)
    b2 = (c2b[0, _IDX2] * _MSK2).reshape(1, 128)
    return b1, b2


def kernel(x, c1w, c1b, c2w, c2b, f1w, f1b, f2w, f2b, f3w, f3b):
    B = x.shape[0]
    n_blocks = max(2, (B + TB - 1) // TB)
    Bp = n_blocks * TB
    xt = jnp.transpose(x, (2, 0, 1, 3)).reshape(32, B, 96).astype(jnp.bfloat16)
    xt = jnp.pad(xt, ((0, 0), (0, Bp - B), (0, 32)))
    w1, w2, f1 = _pack_weights(c1w, c2w, f1w)
    b1v, b2v = _pack_biases(c1b, c2b)

    const2 = lambda i: (0, 0)
    const3 = lambda i: (0, 0, 0)
    logits = pl.pallas_call(
        _lenet_kernel,
        out_shape=jax.ShapeDtypeStruct((Bp, 128), jnp.float32),
        grid=(n_blocks,),
        in_specs=[
            pl.BlockSpec((32, TB, 128), lambda i: (0, i, 0)),
            pl.BlockSpec((640, 256), const2),
            pl.BlockSpec((1, 128), const2),
            pl.BlockSpec((640, 256), const2),
            pl.BlockSpec((1, 128), const2),
            pl.BlockSpec((640, 128), const2),
            pl.BlockSpec((1, 128), const2),
            pl.BlockSpec((128, 128), const2),
            pl.BlockSpec((1, 128), const2),
            pl.BlockSpec((128, 128), const2),
            pl.BlockSpec((1, 128), const2),
        ],
        out_specs=pl.BlockSpec((TB, 128), lambda i: (i, 0)),
        compiler_params=pltpu.CompilerParams(
            dimension_semantics=("parallel",),
            vmem_limit_bytes=48 * 1024 * 1024),
    )(xt, w1, b1v, w2, b2v, f1, f1b, f2w, f2b, f3w, f3b)
    return logits[:B, :10]


# speedup vs baseline: 19.1346x; 1.1606x over previous
import numpy as np
import jax
import jax.numpy as jnp
from jax.experimental import pallas as pl
from jax.experimental.pallas import tpu as pltpu

TB = 256
LANES = 128


def _band(kw_max, w_max, ow_max):
    t = np.zeros((kw_max, w_max, ow_max), np.float32)
    for kw in range(kw_max):
        for ow in range(ow_max):
            t[kw, ow + kw, ow] = 1.0
    return t

_T1 = _band(5, 32, 28)
_T2 = _band(5, 14, 10)

_IDX1 = (np.arange(128) % 6).astype(np.int32)
_MSK1 = (np.arange(128) < 14 * 6).astype(np.float32)
_IDX2 = (np.arange(128) % 16).astype(np.int32)
_MSK2 = (np.arange(128) < 5 * 16).astype(np.float32)


def _lenet_kernel(xt_ref, w1_ref, b1_ref, w2_ref, b2_ref,
                  f1_ref, f1b_ref, f2_ref, f2b_ref, f3_ref, f3b_ref, o_ref):
    tb = o_ref.shape[0]

    xv = jnp.pad(xt_ref[...], ((0, 0), (0, 0), (0, 32)))
    lhs = jnp.concatenate(
        [xv[kh:kh + 28].reshape(tb * 28, LANES) for kh in range(5)], axis=-1)
    m1 = tb * 14
    d0 = jnp.dot(lhs[:m1], w1_ref[...], preferred_element_type=jnp.float32)
    d1 = jnp.dot(lhs[m1:], w1_ref[...], preferred_element_type=jnp.float32)
    y = jnp.concatenate([d0, d1], axis=0).reshape(14, 2, tb, 256)
    y = jnp.maximum(y[:, 0], y[:, 1])
    y = jnp.maximum(y[..., :LANES], y[..., LANES:])
    h1 = jnp.maximum(y + b1_ref[...], 0.0).astype(jnp.bfloat16)

    lhs = jnp.concatenate(
        [h1[kh:kh + 10].reshape(tb * 10, LANES) for kh in range(5)], axis=-1)
    m2 = tb * 5
    d0 = jnp.dot(lhs[:m2], w2_ref[...], preferred_element_type=jnp.float32)
    d1 = jnp.dot(lhs[m2:], w2_ref[...], preferred_element_type=jnp.float32)
    y = jnp.concatenate([d0, d1], axis=0).reshape(5, 2, tb, 256)
    y = jnp.maximum(y[:, 0], y[:, 1])
    y = jnp.maximum(y[..., :LANES], y[..., LANES:])
    h2 = jnp.maximum(y + b2_ref[...], 0.0).astype(jnp.bfloat16)

    lhs = jnp.concatenate([h2[j] for j in range(5)], axis=-1)
    h = jnp.dot(lhs, f1_ref[...], preferred_element_type=jnp.float32)
    h = jnp.maximum(h + f1b_ref[...], 0.0).astype(jnp.bfloat16)
    h = jnp.dot(h, f2_ref[...], preferred_element_type=jnp.float32)
    h = jnp.maximum(h + f2b_ref[...], 0.0).astype(jnp.bfloat16)
    h = jnp.dot(h, f3_ref[...], preferred_element_type=jnp.float32)
    o_ref[...] = h + f3b_ref[...]


def _pack_weights(c1w, c2w, f1w):
    r1 = c1w[:75, :6].astype(jnp.float32).reshape(5, 5, 3, 6)
    w1 = jnp.einsum('xwo,kxcn->kcwon', _T1, r1)
    w1 = w1.reshape(5, 3, 32, 14, 2, 6)
    w1 = jnp.transpose(w1, (0, 1, 2, 4, 3, 5)).reshape(5, 96, 2, 84)
    w1 = jnp.pad(w1, ((0, 0), (0, 0), (0, 0), (0, 128 - 84)))
    w1 = w1.reshape(5, 96, 256)
    w1 = jnp.pad(w1, ((0, 0), (0, 32), (0, 0)))
    w1 = w1.reshape(640, 256).astype(jnp.bfloat16)
    r2 = c2w[:, :, :6, :16].astype(jnp.float32)
    w2 = jnp.einsum('xwo,kxcn->kwcon', _T2, r2)
    w2 = w2.reshape(5, 14, 6, 5, 2, 16)
    w2 = jnp.transpose(w2, (0, 1, 2, 4, 3, 5)).reshape(5, 84, 2, 80)
    w2 = jnp.pad(w2, ((0, 0), (0, 128 - 84), (0, 0), (0, 128 - 80)))
    w2 = w2.reshape(640, 256).astype(jnp.bfloat16)
    f1 = f1w.reshape(5, 5, 128, 128)[:, :, :16, :].reshape(5, 80, 128)
    f1 = jnp.pad(f1, ((0, 0), (0, 128 - 80), (0, 0))).reshape(640, 128)
    return w1, w2, f1


def _pack_biases(c1b, c2b):
    b1 = (c1b[0, _IDX1] * _MSK1).reshape(1, 128)
    b2 = (c2b[0, _IDX2] * _MSK2).reshape(1, 128)
    return b1, b2


def kernel(x, c1w, c1b, c2w, c2b, f1w, f1b, f2w, f2b, f3w, f3b):
    B = x.shape[0]
    n_blocks = max(2, (B + TB - 1) // TB)
    Bp = n_blocks * TB
    xt = jnp.transpose(x, (2, 0, 1, 3)).reshape(32, B, 96).astype(jnp.bfloat16)
    if Bp != B:
        xt = jnp.pad(xt, ((0, 0), (0, Bp - B), (0, 0)))
    w1, w2, f1 = _pack_weights(c1w, c2w, f1w)
    b1v, b2v = _pack_biases(c1b, c2b)

    const2 = lambda i: (0, 0)
    const3 = lambda i: (0, 0, 0)
    logits = pl.pallas_call(
        _lenet_kernel,
        out_shape=jax.ShapeDtypeStruct((Bp, 128), jnp.float32),
        grid=(n_blocks,),
        in_specs=[
            pl.BlockSpec((32, TB, 96), lambda i: (0, i, 0)),
            pl.BlockSpec((640, 256), const2),
            pl.BlockSpec((1, 128), const2),
            pl.BlockSpec((640, 256), const2),
            pl.BlockSpec((1, 128), const2),
            pl.BlockSpec((640, 128), const2),
            pl.BlockSpec((1, 128), const2),
            pl.BlockSpec((128, 128), const2),
            pl.BlockSpec((1, 128), const2),
            pl.BlockSpec((128, 128), const2),
            pl.BlockSpec((1, 128), const2),
        ],
        out_specs=pl.BlockSpec((TB, 128), lambda i: (i, 0)),
        compiler_params=pltpu.CompilerParams(
            dimension_semantics=("parallel",),
            vmem_limit_bytes=48 * 1024 * 1024),
    )(xt, w1, b1v, w2, b2v, f1, f1b, f2w, f2b, f3w, f3b)
    return logits[:B, :10]


# speedup vs baseline: 19.4223x; 1.0150x over previous
import numpy as np
import jax
import jax.numpy as jnp
from jax.experimental import pallas as pl
from jax.experimental.pallas import tpu as pltpu

TB = 512
LANES = 128


def _band(kw_max, w_max, ow_max):
    t = np.zeros((kw_max, w_max, ow_max), np.float32)
    for kw in range(kw_max):
        for ow in range(ow_max):
            t[kw, ow + kw, ow] = 1.0
    return t

_T1 = _band(5, 32, 28)
_T2 = _band(5, 14, 10)

_IDX1 = (np.arange(128) % 6).astype(np.int32)
_MSK1 = (np.arange(128) < 14 * 6).astype(np.float32)
_IDX2 = (np.arange(128) % 16).astype(np.int32)
_MSK2 = (np.arange(128) < 5 * 16).astype(np.float32)


def _lenet_kernel(xt_ref, w1_ref, b1_ref, w2_ref, b2_ref,
                  f1_ref, f1b_ref, f2_ref, f2b_ref, f3_ref, f3b_ref, o_ref):
    tb = o_ref.shape[0]

    xv = jnp.pad(xt_ref[...], ((0, 0), (0, 0), (0, 32)))
    lhs = jnp.concatenate(
        [xv[kh:kh + 28].reshape(tb * 28, LANES) for kh in range(5)], axis=-1)
    m1 = tb * 14
    d0 = jnp.dot(lhs[:m1], w1_ref[...], preferred_element_type=jnp.float32)
    d1 = jnp.dot(lhs[m1:], w1_ref[...], preferred_element_type=jnp.float32)
    y = jnp.concatenate([d0, d1], axis=0).reshape(14, 2, tb, 256)
    y = jnp.maximum(y[:, 0], y[:, 1])
    y = jnp.maximum(y[..., :LANES], y[..., LANES:])
    h1 = jnp.maximum(y + b1_ref[...], 0.0).astype(jnp.bfloat16)

    lhs = jnp.concatenate(
        [h1[kh:kh + 10].reshape(tb * 10, LANES) for kh in range(5)], axis=-1)
    m2 = tb * 5
    d0 = jnp.dot(lhs[:m2], w2_ref[...], preferred_element_type=jnp.float32)
    d1 = jnp.dot(lhs[m2:], w2_ref[...], preferred_element_type=jnp.float32)
    y = jnp.concatenate([d0, d1], axis=0).reshape(5, 2, tb, 256)
    y = jnp.maximum(y[:, 0], y[:, 1])
    y = jnp.maximum(y[..., :LANES], y[..., LANES:])
    h2 = jnp.maximum(y + b2_ref[...], 0.0).astype(jnp.bfloat16)

    lhs = jnp.concatenate([h2[j] for j in range(5)], axis=-1)
    h = jnp.dot(lhs, f1_ref[...], preferred_element_type=jnp.float32)
    h = jnp.maximum(h + f1b_ref[...], 0.0).astype(jnp.bfloat16)
    h = jnp.dot(h, f2_ref[...], preferred_element_type=jnp.float32)
    h = jnp.maximum(h + f2b_ref[...], 0.0).astype(jnp.bfloat16)
    h = jnp.dot(h, f3_ref[...], preferred_element_type=jnp.float32)
    o_ref[...] = h + f3b_ref[...]


def _pack_weights(c1w, c2w, f1w):
    r1 = c1w[:75, :6].astype(jnp.float32).reshape(5, 5, 3, 6)
    w1 = jnp.einsum('xwo,kxcn->kcwon', _T1, r1)
    w1 = w1.reshape(5, 3, 32, 14, 2, 6)
    w1 = jnp.transpose(w1, (0, 1, 2, 4, 3, 5)).reshape(5, 96, 2, 84)
    w1 = jnp.pad(w1, ((0, 0), (0, 0), (0, 0), (0, 128 - 84)))
    w1 = w1.reshape(5, 96, 256)
    w1 = jnp.pad(w1, ((0, 0), (0, 32), (0, 0)))
    w1 = w1.reshape(640, 256).astype(jnp.bfloat16)
    r2 = c2w[:, :, :6, :16].astype(jnp.float32)
    w2 = jnp.einsum('xwo,kxcn->kwcon', _T2, r2)
    w2 = w2.reshape(5, 14, 6, 5, 2, 16)
    w2 = jnp.transpose(w2, (0, 1, 2, 4, 3, 5)).reshape(5, 84, 2, 80)
    w2 = jnp.pad(w2, ((0, 0), (0, 128 - 84), (0, 0), (0, 128 - 80)))
    w2 = w2.reshape(640, 256).astype(jnp.bfloat16)
    f1 = f1w.reshape(5, 5, 128, 128)[:, :, :16, :].reshape(5, 80, 128)
    f1 = jnp.pad(f1, ((0, 0), (0, 128 - 80), (0, 0))).reshape(640, 128)
    return w1, w2, f1


def _pack_biases(c1b, c2b):
    b1 = (c1b[0, _IDX1] * _MSK1).reshape(1, 128)
    b2 = (c2b[0, _IDX2] * _MSK2).reshape(1, 128)
    return b1, b2


def kernel(x, c1w, c1b, c2w, c2b, f1w, f1b, f2w, f2b, f3w, f3b):
    B = x.shape[0]
    n_blocks = max(2, (B + TB - 1) // TB)
    Bp = n_blocks * TB
    xt = jnp.transpose(x, (2, 0, 1, 3)).reshape(32, B, 96).astype(jnp.bfloat16)
    if Bp != B:
        xt = jnp.pad(xt, ((0, 0), (0, Bp - B), (0, 0)))
    w1, w2, f1 = _pack_weights(c1w, c2w, f1w)
    b1v, b2v = _pack_biases(c1b, c2b)

    const2 = lambda i: (0, 0)
    const3 = lambda i: (0, 0, 0)
    logits = pl.pallas_call(
        _lenet_kernel,
        out_shape=jax.ShapeDtypeStruct((Bp, 128), jnp.float32),
        grid=(n_blocks,),
        in_specs=[
            pl.BlockSpec((32, TB, 96), lambda i: (0, i, 0)),
            pl.BlockSpec((640, 256), const2),
            pl.BlockSpec((1, 128), const2),
            pl.BlockSpec((640, 256), const2),
            pl.BlockSpec((1, 128), const2),
            pl.BlockSpec((640, 128), const2),
            pl.BlockSpec((1, 128), const2),
            pl.BlockSpec((128, 128), const2),
            pl.BlockSpec((1, 128), const2),
            pl.BlockSpec((128, 128), const2),
            pl.BlockSpec((1, 128), const2),
        ],
        out_specs=pl.BlockSpec((TB, 128), lambda i: (i, 0)),
        compiler_params=pltpu.CompilerParams(
            dimension_semantics=("parallel",),
            vmem_limit_bytes=48 * 1024 * 1024),
    )(xt, w1, b1v, w2, b2v, f1, f1b, f2w, f2b, f3w, f3b)
    return logits[:B, :10]


# speedup vs baseline: 19.4752x; 1.0027x over previous
import numpy as np
import jax
import jax.numpy as jnp
from jax.experimental import pallas as pl
from jax.experimental.pallas import tpu as pltpu

TB = 512
LANES = 128


def _band(kw_max, w_max, ow_max):
    t = np.zeros((kw_max, w_max, ow_max), np.float32)
    for kw in range(kw_max):
        for ow in range(ow_max):
            t[kw, ow + kw, ow] = 1.0
    return t

_T1 = _band(5, 32, 28)
_T2 = _band(5, 14, 10)

_IDX1 = (np.arange(128) % 6).astype(np.int32)
_MSK1 = (np.arange(128) < 14 * 6).astype(np.float32)
_IDX2 = (np.arange(128) % 16).astype(np.int32)
_MSK2 = (np.arange(128) < 5 * 16).astype(np.float32)


def _lenet_kernel(xt_ref, w1_ref, b1_ref, w2_ref, b2_ref,
                  f1_ref, f1b_ref, f2_ref, f2b_ref, f3_ref, f3b_ref, o_ref):
    tb = o_ref.shape[0]

    xv = jnp.pad(xt_ref[...], ((0, 0), (0, 0), (0, 32)))
    lhs = jnp.concatenate(
        [xv[kh:kh + 28].reshape(tb * 28, LANES) for kh in range(5)], axis=-1)
    m1 = tb * 14
    d0 = jnp.dot(lhs[:m1], w1_ref[...], preferred_element_type=jnp.float32)
    d1 = jnp.dot(lhs[m1:], w1_ref[...], preferred_element_type=jnp.float32)
    y = jnp.concatenate([d0, d1], axis=0).reshape(14, 2, tb, 256)
    y = jnp.maximum(y[:, 0], y[:, 1])
    y = jnp.maximum(y[..., :LANES], y[..., LANES:])
    h1 = jnp.maximum(y + b1_ref[...], 0.0).astype(jnp.bfloat16)

    lhs = jnp.concatenate(
        [h1[kh:kh + 10].reshape(tb * 10, LANES) for kh in range(5)], axis=-1)
    m2 = tb * 5
    d0 = jnp.dot(lhs[:m2], w2_ref[...], preferred_element_type=jnp.float32)
    d1 = jnp.dot(lhs[m2:], w2_ref[...], preferred_element_type=jnp.float32)
    y = jnp.concatenate([d0, d1], axis=0).reshape(5, 2, tb, 256)
    y = jnp.maximum(y[:, 0], y[:, 1])
    y = jnp.maximum(y[..., :LANES], y[..., LANES:])
    h2 = jnp.maximum(y + b2_ref[...], 0.0).astype(jnp.bfloat16)

    lhs = jnp.concatenate([h2[j] for j in range(5)], axis=-1)
    h = jnp.dot(lhs, f1_ref[...], preferred_element_type=jnp.float32)
    h = jnp.maximum(h + f1b_ref[...], 0.0).astype(jnp.bfloat16)
    h = jnp.dot(h, f2_ref[...], preferred_element_type=jnp.float32)
    h = jnp.maximum(h + f2b_ref[...], 0.0).astype(jnp.bfloat16)
    h = jnp.dot(h, f3_ref[...], preferred_element_type=jnp.float32)
    o_ref[...] = h + f3b_ref[...]


def _pack_weights(c1w, c2w, f1w):
    r1 = c1w[:75, :6].astype(jnp.float32).reshape(5, 5, 3, 6)
    w1 = jnp.einsum('xwo,kxcn->kcwon', _T1, r1)
    w1 = w1.reshape(5, 3, 32, 14, 2, 6)
    w1 = jnp.transpose(w1, (0, 1, 2, 4, 3, 5)).reshape(5, 96, 2, 84)
    w1 = jnp.pad(w1, ((0, 0), (0, 0), (0, 0), (0, 128 - 84)))
    w1 = w1.reshape(5, 96, 256)
    w1 = jnp.pad(w1, ((0, 0), (0, 32), (0, 0)))
    w1 = w1.reshape(640, 256).astype(jnp.bfloat16)
    r2 = c2w[:, :, :6, :16].astype(jnp.float32)
    w2 = jnp.einsum('xwo,kxcn->kwcon', _T2, r2)
    w2 = w2.reshape(5, 14, 6, 5, 2, 16)
    w2 = jnp.transpose(w2, (0, 1, 2, 4, 3, 5)).reshape(5, 84, 2, 80)
    w2 = jnp.pad(w2, ((0, 0), (0, 128 - 84), (0, 0), (0, 128 - 80)))
    w2 = w2.reshape(640, 256).astype(jnp.bfloat16)
    f1 = f1w.reshape(5, 5, 128, 128)[:, :, :16, :].reshape(5, 80, 128)
    f1 = jnp.pad(f1, ((0, 0), (0, 128 - 80), (0, 0))).reshape(640, 128)
    return w1, w2, f1


def _pack_biases(c1b, c2b):
    b1 = (c1b[0, _IDX1] * _MSK1).reshape(1, 128)
    b2 = (c2b[0, _IDX2] * _MSK2).reshape(1, 128)
    return b1, b2


def kernel(x, c1w, c1b, c2w, c2b, f1w, f1b, f2w, f2b, f3w, f3b):
    B = x.shape[0]
    n_blocks = max(2, (B + TB - 1) // TB)
    Bp = n_blocks * TB
    xt = jnp.transpose(x, (2, 0, 1, 3)).reshape(32, B, 96).astype(jnp.bfloat16)
    if Bp != B:
        xt = jnp.pad(xt, ((0, 0), (0, Bp - B), (0, 0)))
    w1, w2, f1 = _pack_weights(c1w, c2w, f1w)
    b1v, b2v = _pack_biases(c1b, c2b)

    const2 = lambda i: (0, 0)
    logits = pl.pallas_call(
        _lenet_kernel,
        out_shape=jax.ShapeDtypeStruct((Bp, 128), jnp.float32),
        grid=(n_blocks,),
        in_specs=[
            pl.BlockSpec((32, TB, 96), lambda i: (0, i, 0)),
            pl.BlockSpec((640, 256), const2),
            pl.BlockSpec((1, 128), const2),
            pl.BlockSpec((640, 256), const2),
            pl.BlockSpec((1, 128), const2),
            pl.BlockSpec((640, 128), const2),
            pl.BlockSpec((1, 128), const2),
            pl.BlockSpec((128, 128), const2),
            pl.BlockSpec((1, 128), const2),
            pl.BlockSpec((128, 128), const2),
            pl.BlockSpec((1, 128), const2),
        ],
        out_specs=pl.BlockSpec((TB, 128), lambda i: (i, 0)),
        compiler_params=pltpu.CompilerParams(
            dimension_semantics=("parallel",),
            vmem_limit_bytes=48 * 1024 * 1024),
    )(xt, w1, b1v, w2, b2v, f1, f1b, f2w, f2b, f3w, f3b)
    return logits[:B, :10]
